# Initial kernel scaffold; baseline (speedup 1.0000x reference)
import jax
import jax.numpy as jnp
from jax.experimental import pallas as pl


def kernel(x, w1, b1, g1, be1, w2, b2, g2, be2, w3, b3, wd, bd):
    raise NotImplementedError("write your pallas kernel here")


# speedup vs baseline: 1.0797x; 1.0797x over previous
import functools

import jax
import jax.numpy as jnp
from jax import lax
from jax.experimental import pallas as pl
from jax.experimental.pallas import tpu as pltpu

_LN_EPS = 1e-5
_INV_SQRT2 = 0.7071067811865475244
_PACK = 8


def _cdiv(a, b):
    return (a + b - 1) // b


def _gelu_exact(x):
    return 0.5 * x * (1.0 + lax.erf(x * _INV_SQRT2))


def _judge_kernel(x_ref, w1_ref, w2_ref, seg_ref, g8_ref,
                  b1_ref, g1_ref, be1_ref, b2_ref,
                  ut_ref, wdt_ref, c8_ref, o_ref):
    a = x_ref[...]
    seg = seg_ref[...]

    h = jnp.dot(a, w1_ref[...], preferred_element_type=jnp.float32)
    h = _gelu_exact(h + b1_ref[...])

    s1 = jnp.dot(jnp.concatenate([h, h * h], axis=0), seg,
                 preferred_element_type=jnp.float32)
    tm = a.shape[0]
    mu1 = s1[:tm]
    var1 = s1[tm:] - mu1 * mu1
    ln1 = (h - mu1) * lax.rsqrt(var1 + _LN_EPS) * g1_ref[...] + be1_ref[...]

    h2 = jnp.dot(ln1, w2_ref[...], preferred_element_type=jnp.float32)
    h2 = _gelu_exact(h2 + b2_ref[...])

    ut = ut_ref[...]
    s2 = jnp.dot(jnp.concatenate([h2, h2 * h2, h2 * ut], axis=0), seg,
                 preferred_element_type=jnp.float32)
    mu2 = s2[:tm]
    var2 = s2[tm:2 * tm] - mu2 * mu2
    wavg = s2[2 * tm:]
    istd2 = lax.rsqrt(var2 + _LN_EPS)

    ubar = jnp.sum(ut[:, :32], dtype=jnp.float32) * (1.0 / 32.0)
    lhs3 = istd2 * (wavg - mu2 * ubar) + a * wdt_ref[...]
    c = jnp.dot(lhs3, g8_ref[...], preferred_element_type=jnp.float32)
    o_ref[...] = c + c8_ref[...]


def _judge(x, w1, b1, g1, be1, w2, b2, g2, be2, w3, b3, wd, bd,
           *, tile_rows=2048):
    in_places, hidden = w1.shape
    lead = x.shape[:-1]
    xf = x.reshape(-1, in_places)
    n = xf.shape[0]
    f32 = jnp.float32

    wx = _PACK * in_places
    n_rows = _cdiv(n, _PACK)

    tm = max(8, min(int(tile_rows), ((n_rows + 1) // 2 + 7) // 8 * 8))
    grid_n = _cdiv(n_rows, tm)
    n_rows_pad = grid_n * tm
    n_pad = n_rows_pad * _PACK
    if n_pad != n:
        xf = jnp.pad(xf, ((0, n_pad - n), (0, 0)))
    xp = xf.reshape(n_rows_pad, wx).astype(f32)

    eye8 = jnp.eye(_PACK, dtype=f32)
    w1b = jnp.kron(eye8, w1.astype(f32))
    w2b = jnp.kron(eye8, w2.astype(f32))
    seg = jnp.kron(eye8, jnp.full((in_places, hidden), 1.0 / hidden, f32))
    g8 = jnp.kron(eye8, jnp.ones((hidden, 1), f32))

    tile = lambda v: jnp.tile(v.reshape(1, -1).astype(f32), (1, _PACK))
    b1t, g1t, be1t, b2t = tile(b1), tile(g1), tile(be1), tile(b2)
    ut = tile(g2.reshape(-1) * w3.reshape(-1))
    wdt = tile(wd.reshape(-1))
    cscal = (jnp.sum(be2.reshape(-1).astype(f32) * w3.reshape(-1).astype(f32))
             + b3.reshape(()).astype(f32) + bd.reshape(()).astype(f32))
    c8 = jnp.full((1, _PACK), 1.0, f32) * cscal

    const = lambda shape: pl.BlockSpec(shape, lambda i: (0,) * len(shape))
    est_vmem = (4 * tm * wx * 4) * 12 + 4 * (wx * wx * 3)
    vmem_limit = int(min(max(est_vmem, 16 << 20), 110 << 20))

    out = pl.pallas_call(
        _judge_kernel,
        out_shape=jax.ShapeDtypeStruct((n_rows_pad, _PACK), f32),
        grid=(grid_n,),
        in_specs=[pl.BlockSpec((tm, wx), lambda i: (i, 0)),
                  const((wx, wx)), const((wx, wx)), const((wx, wx)),
                  const((wx, _PACK)),
                  const((1, wx)), const((1, wx)), const((1, wx)),
                  const((1, wx)), const((1, wx)), const((1, wx)),
                  const((1, _PACK))],
        out_specs=pl.BlockSpec((tm, _PACK), lambda i: (i, 0)),
        compiler_params=pltpu.CompilerParams(
            dimension_semantics=("parallel",),
            vmem_limit_bytes=vmem_limit),
    )(xp, w1b, w2b, seg, g8, b1t, g1t, be1t, b2t, ut, wdt, c8)

    out = out.reshape(n_pad, 1)[:n]
    return out.reshape(*lead, 1).astype(x.dtype)


def kernel(x, w1, b1, g1, be1, w2, b2, g2, be2, w3, b3, wd, bd):
    return _judge(x, w1, b1, g1, be1, w2, b2, g2, be2, w3, b3, wd, bd)


# speedup vs baseline: 1.5875x; 1.4703x over previous
import functools

import jax
import jax.numpy as jnp
from jax import lax
from jax.experimental import pallas as pl
from jax.experimental.pallas import tpu as pltpu

_LN_EPS = 1e-5
_INV_SQRT2 = 0.7071067811865475244
_PACK = 8


def _cdiv(a, b):
    return (a + b - 1) // b


def _gelu_exact(x):
    return 0.5 * x * (1.0 + lax.erf(x * _INV_SQRT2))


def _judge_kernel(x_ref, w1_ref, w2_ref, seg_ref, g8_ref,
                  b1_ref, g1_ref, be1_ref, b2_ref,
                  ut_ref, wdt_ref, c8_ref, o_ref):
    x3 = x_ref[...]
    a = x3.reshape(x3.shape[0], 256)
    seg = seg_ref[...]

    h = jnp.dot(a, w1_ref[...], preferred_element_type=jnp.float32)
    h = _gelu_exact(h + b1_ref[...])

    s1 = jnp.dot(jnp.concatenate([h, h * h], axis=0), seg,
                 preferred_element_type=jnp.float32)
    tm = a.shape[0]
    mu1 = s1[:tm]
    var1 = s1[tm:] - mu1 * mu1
    ln1 = (h - mu1) * lax.rsqrt(var1 + _LN_EPS) * g1_ref[...] + be1_ref[...]

    h2 = jnp.dot(ln1, w2_ref[...], preferred_element_type=jnp.float32)
    h2 = _gelu_exact(h2 + b2_ref[...])

    ut = ut_ref[...]
    s2 = jnp.dot(jnp.concatenate([h2, h2 * h2, h2 * ut], axis=0), seg,
                 preferred_element_type=jnp.float32)
    mu2 = s2[:tm]
    var2 = s2[tm:2 * tm] - mu2 * mu2
    wavg = s2[2 * tm:]
    istd2 = lax.rsqrt(var2 + _LN_EPS)

    ubar = jnp.sum(ut[:, :32], dtype=jnp.float32) * (1.0 / 32.0)
    lhs3 = istd2 * (wavg - mu2 * ubar) + a * wdt_ref[...]
    c = jnp.dot(lhs3, g8_ref[...], preferred_element_type=jnp.float32)
    o_ref[...] = c + c8_ref[...]


def _judge(x, w1, b1, g1, be1, w2, b2, g2, be2, w3, b3, wd, bd,
           *, tile_rows=2048):
    in_places, hidden = w1.shape
    lead = x.shape[:-1]
    xf = x.reshape(-1, in_places)
    n = xf.shape[0]
    f32 = jnp.float32

    wx = _PACK * in_places
    n_rows = _cdiv(n, _PACK)

    tm = max(8, min(int(tile_rows), ((n_rows + 1) // 2 + 7) // 8 * 8))
    grid_n = _cdiv(n_rows, tm)
    n_rows_pad = grid_n * tm
    n_pad = n_rows_pad * _PACK
    if n_pad != n:
        xf = jnp.pad(xf, ((0, n_pad - n), (0, 0)))
    xp = xf.reshape(n_rows_pad, _PACK, in_places).astype(f32)

    eye8 = jnp.eye(_PACK, dtype=f32)
    w1b = jnp.kron(eye8, w1.astype(f32))
    w2b = jnp.kron(eye8, w2.astype(f32))
    seg = jnp.kron(eye8, jnp.full((in_places, hidden), 1.0 / hidden, f32))
    g8 = jnp.kron(eye8, jnp.ones((hidden, 1), f32))

    tile = lambda v: jnp.tile(v.reshape(1, -1).astype(f32), (1, _PACK))
    b1t, g1t, be1t, b2t = tile(b1), tile(g1), tile(be1), tile(b2)
    ut = tile(g2.reshape(-1) * w3.reshape(-1))
    wdt = tile(wd.reshape(-1))
    cscal = (jnp.sum(be2.reshape(-1).astype(f32) * w3.reshape(-1).astype(f32))
             + b3.reshape(()).astype(f32) + bd.reshape(()).astype(f32))
    c8 = jnp.full((1, _PACK), 1.0, f32) * cscal

    const = lambda shape: pl.BlockSpec(shape, lambda i: (0,) * len(shape))
    est_vmem = (4 * tm * wx * 4) * 12 + 4 * (wx * wx * 3)
    vmem_limit = int(min(max(est_vmem, 16 << 20), 110 << 20))

    out = pl.pallas_call(
        _judge_kernel,
        out_shape=jax.ShapeDtypeStruct((n_rows_pad, _PACK), f32),
        grid=(grid_n,),
        in_specs=[pl.BlockSpec((tm, _PACK, in_places), lambda i: (i, 0, 0)),
                  const((wx, wx)), const((wx, wx)), const((wx, wx)),
                  const((wx, _PACK)),
                  const((1, wx)), const((1, wx)), const((1, wx)),
                  const((1, wx)), const((1, wx)), const((1, wx)),
                  const((1, _PACK))],
        out_specs=pl.BlockSpec((tm, _PACK), lambda i: (i, 0)),
        compiler_params=pltpu.CompilerParams(
            dimension_semantics=("parallel",),
            vmem_limit_bytes=vmem_limit),
    )(xp, w1b, w2b, seg, g8, b1t, g1t, be1t, b2t, ut, wdt, c8)

    out = out.reshape(n_pad, 1)[:n]
    return out.reshape(*lead, 1).astype(x.dtype)


def kernel(x, w1, b1, g1, be1, w2, b2, g2, be2, w3, b3, wd, bd):
    return _judge(x, w1, b1, g1, be1, w2, b2, g2, be2, w3, b3, wd, bd)


# speedup vs baseline: 1.8098x; 1.1401x over previous
import functools

import jax
import jax.numpy as jnp
from jax import lax
from jax.experimental import pallas as pl
from jax.experimental.pallas import tpu as pltpu

_LN_EPS = 1e-5
_INV_SQRT2 = 0.7071067811865475244
_PACK = 8


def _cdiv(a, b):
    return (a + b - 1) // b


def _gelu2(x):
    return x * (1.0 + lax.erf(x * _INV_SQRT2))


def _judge_kernel(x_ref, w1_ref, w2_ref, seg_ref, g8_ref,
                  b1_ref, b2_ref,
                  ut_ref, wdt_ref, c8_ref, o_ref):
    x3 = x_ref[...]
    a = x3.reshape(x3.shape[0], 256)
    seg = seg_ref[...]
    eps4 = 4.0 * _LN_EPS

    h = jnp.dot(a, w1_ref[...], preferred_element_type=jnp.float32)
    h = _gelu2(h + b1_ref[...])

    mu1 = jnp.dot(h, seg, preferred_element_type=jnp.float32)
    m2 = jnp.dot(h * h, seg, preferred_element_type=jnp.float32)
    cs1 = (h - mu1) * lax.rsqrt(m2 - mu1 * mu1 + eps4)

    h2 = jnp.dot(cs1, w2_ref[...], preferred_element_type=jnp.float32)
    h2 = _gelu2(h2 + b2_ref[...])

    ut = ut_ref[...]
    mu2 = jnp.dot(h2, seg, preferred_element_type=jnp.float32)
    m22 = jnp.dot(h2 * h2, seg, preferred_element_type=jnp.float32)
    wavg = jnp.dot(h2 * ut, seg, preferred_element_type=jnp.float32)
    istd2 = lax.rsqrt(m22 - mu2 * mu2 + eps4)

    ubar = jnp.sum(ut, dtype=jnp.float32) * (1.0 / 256.0)
    lhs3 = istd2 * (wavg - mu2 * ubar) + a * wdt_ref[...]
    c = jnp.dot(lhs3, g8_ref[...], preferred_element_type=jnp.float32)
    o_ref[...] = c + c8_ref[...]


def _judge(x, w1, b1, g1, be1, w2, b2, g2, be2, w3, b3, wd, bd,
           *, tile_rows=2048):
    in_places, hidden = w1.shape
    lead = x.shape[:-1]
    xf = x.reshape(-1, in_places)
    n = xf.shape[0]
    f32 = jnp.float32

    wx = _PACK * in_places
    n_rows = _cdiv(n, _PACK)

    tm = max(8, min(int(tile_rows), ((n_rows + 1) // 2 + 7) // 8 * 8))
    grid_n = _cdiv(n_rows, tm)
    n_rows_pad = grid_n * tm
    n_pad = n_rows_pad * _PACK
    if n_pad != n:
        xf = jnp.pad(xf, ((0, n_pad - n), (0, 0)))
    xp = xf.reshape(n_rows_pad, _PACK, in_places).astype(f32)

    eye8 = jnp.eye(_PACK, dtype=f32)
    w1b = jnp.kron(eye8, w1.astype(f32))
    w2b = jnp.kron(eye8, w2.astype(f32))
    seg = jnp.kron(eye8, jnp.full((in_places, hidden), 1.0 / hidden, f32))
    g8 = jnp.kron(eye8, jnp.ones((hidden, 1), f32))

    tile = lambda v: jnp.tile(v.reshape(1, -1).astype(f32), (1, _PACK))
    b1t, b2t = tile(b1), tile(b2)
    g1t, be1t = tile(g1), tile(be1)
    b2t = b2t + be1t @ w2b
    w2b = g1t.reshape(-1, 1) * w2b
    ut = tile(g2.reshape(-1) * w3.reshape(-1))
    wdt = tile(wd.reshape(-1))
    cscal = (jnp.sum(be2.reshape(-1).astype(f32) * w3.reshape(-1).astype(f32))
             + b3.reshape(()).astype(f32) + bd.reshape(()).astype(f32))
    c8 = jnp.full((1, _PACK), 1.0, f32) * cscal

    const = lambda shape: pl.BlockSpec(shape, lambda i: (0,) * len(shape))
    est_vmem = (4 * tm * wx * 4) * 12 + 4 * (wx * wx * 3)
    vmem_limit = int(min(max(est_vmem, 16 << 20), 110 << 20))

    out = pl.pallas_call(
        _judge_kernel,
        out_shape=jax.ShapeDtypeStruct((n_rows_pad, _PACK), f32),
        grid=(grid_n,),
        in_specs=[pl.BlockSpec((tm, _PACK, in_places), lambda i: (i, 0, 0)),
                  const((wx, wx)), const((wx, wx)), const((wx, wx)),
                  const((wx, _PACK)),
                  const((1, wx)), const((1, wx)),
                  const((1, wx)), const((1, wx)),
                  const((1, _PACK))],
        out_specs=pl.BlockSpec((tm, _PACK), lambda i: (i, 0)),
        compiler_params=pltpu.CompilerParams(
            dimension_semantics=("parallel",),
            vmem_limit_bytes=vmem_limit),
    )(xp, w1b, w2b, seg, g8, b1t, b2t, ut, wdt, c8)

    out = out.reshape(n_pad, 1)[:n]
    return out.reshape(*lead, 1).astype(x.dtype)


def kernel(x, w1, b1, g1, be1, w2, b2, g2, be2, w3, b3, wd, bd):
    return _judge(x, w1, b1, g1, be1, w2, b2, g2, be2, w3, b3, wd, bd)


# speedup vs baseline: 1.8174x; 1.0042x over previous
import functools

import jax
import jax.numpy as jnp
from jax import lax
from jax.experimental import pallas as pl
from jax.experimental.pallas import tpu as pltpu

_LN_EPS = 1e-5
_INV_SQRT2 = 0.7071067811865475244
_PACK = 8


def _cdiv(a, b):
    return (a + b - 1) // b


def _gelu2(x):
    return x * (1.0 + lax.erf(x * _INV_SQRT2))


def _judge_kernel(x_ref, w1_ref, w2_ref, seg_ref, g8_ref,
                  b1_ref, b2_ref,
                  ut_ref, wdt_ref, c8_ref, o_ref):
    x3 = x_ref[...]
    a = x3.reshape(x3.shape[0], 256)
    seg = seg_ref[...]
    eps4 = 4.0 * _LN_EPS

    h = jnp.dot(a, w1_ref[...], preferred_element_type=jnp.float32)
    h = _gelu2(h + b1_ref[...])

    mu1 = jnp.dot(h, seg, preferred_element_type=jnp.float32)
    m2 = jnp.dot(h * h, seg, preferred_element_type=jnp.float32)
    cs1 = (h - mu1) * lax.rsqrt(m2 - mu1 * mu1 + eps4)

    h2 = jnp.dot(cs1, w2_ref[...], preferred_element_type=jnp.float32)
    h2 = _gelu2(h2 + b2_ref[...])

    ut = ut_ref[...]
    mu2 = jnp.dot(h2, seg, preferred_element_type=jnp.float32)
    m22 = jnp.dot(h2 * h2, seg, preferred_element_type=jnp.float32)
    wavg = jnp.dot(h2 * ut, seg, preferred_element_type=jnp.float32)
    istd2 = lax.rsqrt(m22 - mu2 * mu2 + eps4)

    ubar = jnp.sum(ut, dtype=jnp.float32) * (1.0 / 256.0)
    lhs3 = istd2 * (wavg - mu2 * ubar) + a * wdt_ref[...]
    c = jnp.dot(lhs3, g8_ref[...], preferred_element_type=jnp.float32)
    o_ref[...] = c + c8_ref[...]


def _judge(x, w1, b1, g1, be1, w2, b2, g2, be2, w3, b3, wd, bd,
           *, tile_rows=4096):
    in_places, hidden = w1.shape
    lead = x.shape[:-1]
    xf = x.reshape(-1, in_places)
    n = xf.shape[0]
    f32 = jnp.float32

    wx = _PACK * in_places
    n_rows = _cdiv(n, _PACK)

    tm = max(16, min(int(tile_rows), ((n_rows + 1) // 2 + 15) // 16 * 16))
    grid_n = _cdiv(n_rows, tm)
    n_rows_pad = grid_n * tm
    n_pad = n_rows_pad * _PACK
    if n_pad != n:
        xf = jnp.pad(xf, ((0, n_pad - n), (0, 0)))
    xp = xf.reshape(n_rows_pad, _PACK, in_places).astype(f32)

    eye8 = jnp.eye(_PACK, dtype=f32)
    w1b = jnp.kron(eye8, w1.astype(f32))
    w2b = jnp.kron(eye8, w2.astype(f32))
    seg = jnp.kron(eye8, jnp.full((in_places, hidden), 1.0 / hidden, f32))
    g8 = jnp.kron(eye8, jnp.ones((hidden, 1), f32))

    tile = lambda v: jnp.tile(v.reshape(1, -1).astype(f32), (1, _PACK))
    b1t, b2t = tile(b1), tile(b2)
    g1t, be1t = tile(g1), tile(be1)
    b2t = b2t + be1t @ w2b
    w2b = g1t.reshape(-1, 1) * w2b
    ut = tile(g2.reshape(-1) * w3.reshape(-1))
    wdt = tile(wd.reshape(-1))
    cscal = (jnp.sum(be2.reshape(-1).astype(f32) * w3.reshape(-1).astype(f32))
             + b3.reshape(()).astype(f32) + bd.reshape(()).astype(f32))
    c8 = jnp.full((1, _PACK), 1.0, f32) * cscal

    const = lambda shape: pl.BlockSpec(shape, lambda i: (0,) * len(shape))
    est_vmem = (4 * tm * wx * 4) * 12 + 4 * (wx * wx * 3)
    vmem_limit = int(min(max(est_vmem, 16 << 20), 110 << 20))

    out = pl.pallas_call(
        _judge_kernel,
        out_shape=jax.ShapeDtypeStruct((n_rows_pad, _PACK), f32),
        grid=(grid_n,),
        in_specs=[pl.BlockSpec((tm, _PACK, in_places), lambda i: (i, 0, 0)),
                  const((wx, wx)), const((wx, wx)), const((wx, wx)),
                  const((wx, _PACK)),
                  const((1, wx)), const((1, wx)),
                  const((1, wx)), const((1, wx)),
                  const((1, _PACK))],
        out_specs=pl.BlockSpec((tm, _PACK), lambda i: (i, 0)),
        compiler_params=pltpu.CompilerParams(
            dimension_semantics=("parallel",),
            vmem_limit_bytes=vmem_limit),
    )(xp, w1b, w2b, seg, g8, b1t, b2t, ut, wdt, c8)

    out = out.reshape(n_pad, 1)[:n]
    return out.reshape(*lead, 1).astype(x.dtype)


def kernel(x, w1, b1, g1, be1, w2, b2, g2, be2, w3, b3, wd, bd):
    return _judge(x, w1, b1, g1, be1, w2, b2, g2, be2, w3, b3, wd, bd)
